# baseline (speedup 1.0000x reference)
import functools

import jax
import jax.numpy as jnp
from jax import lax
from jax.experimental import pallas as pl
from jax.experimental.pallas import tpu as pltpu
from jax.experimental.pallas import tpu_sc as plsc

_GRID = 128
_CELLS = _GRID ** 3
_SCENES = 4
_HIDDEN = 16
_N = 262144
_ROWS = _N // 128
_BLK_ROWS = 256
_DECAY = 0.9

_NW = 32
_SPW = _N // _NW
_NB = 64
_BCELLS = _CELLS // _NB
_BSHIFT = 15
_PAD = 1024
_SLOT = 256


def _part1by2(x):
    x = x & jnp.uint32(0x3FF)
    x = (x | (x << 16)) & jnp.uint32(0x30000FF)
    x = (x | (x << 8)) & jnp.uint32(0x300F00F)
    x = (x | (x << 4)) & jnp.uint32(0x30C30C3)
    x = (x | (x << 2)) & jnp.uint32(0x9249249)
    return x


def _sigma_body(coords_ref, code_ref, Wc_ref, w1_ref, b1_ref, w2_ref, b2_ref,
                idx_ref, s01_ref, s23_ref):
    cx = coords_ref[0]
    cy = coords_ref[1]
    cz = coords_ref[2]
    mx = _part1by2(cx.astype(jnp.uint32))
    my = _part1by2(cy.astype(jnp.uint32))
    mz = _part1by2(cz.astype(jnp.uint32))
    idx_ref[...] = (mx | (my << 1) | (mz << 2)).astype(jnp.int32)
    scale = jnp.float32(2.0 / _GRID)
    half = jnp.float32((_GRID - 1) / 2.0)
    fx = (cx.astype(jnp.float32) - half) * scale
    fy = (cy.astype(jnp.float32) - half) * scale
    fz = (cz.astype(jnp.float32) - half) * scale
    cw = jnp.dot(code_ref[...], Wc_ref[...],
                 preferred_element_type=jnp.float32) + b1_ref[...]
    w1 = w1_ref[...]
    w2 = w2_ref[...]
    accs = [jnp.zeros(fx.shape, jnp.float32) for _ in range(_SCENES)]
    for j in range(_HIDDEN):
        base = fx * w1[0, j] + fy * w1[1, j] + fz * w1[2, j]
        for s in range(_SCENES):
            accs[s] = accs[s] + jnp.maximum(base + cw[s, j], 0.0) * w2[j, 0]
    b2v = b2_ref[0, 0]
    sig = [jax.nn.softplus(a + b2v) for a in accs]
    bits = [jax.lax.bitcast_convert_type(
        s.astype(jnp.bfloat16), jnp.uint16).astype(jnp.uint32) for s in sig]
    s01_ref[...] = bits[0] | (bits[1] << 16)
    s23_ref[...] = bits[2] | (bits[3] << 16)


def _sigma_stage(coords, code, W1, b1, Wc, W2, b2):
    coords3 = coords.T.reshape(3, _ROWS, 128)
    grid = (_ROWS // _BLK_ROWS,)
    full = lambda a: pl.BlockSpec(a.shape, lambda i: tuple(0 for _ in a.shape))
    b1r = b1.reshape(1, _HIDDEN)
    b2r = b2.reshape(1, 1)
    out_shape = [
        jax.ShapeDtypeStruct((_ROWS, 128), jnp.int32),
        jax.ShapeDtypeStruct((_ROWS, 128), jnp.uint32),
        jax.ShapeDtypeStruct((_ROWS, 128), jnp.uint32),
    ]
    idx, s01, s23 = pl.pallas_call(
        _sigma_body,
        grid=grid,
        in_specs=[
            pl.BlockSpec((3, _BLK_ROWS, 128), lambda i: (0, i, 0)),
            full(code), full(Wc), full(W1), full(b1r), full(W2), full(b2r),
        ],
        out_specs=[
            pl.BlockSpec((_BLK_ROWS, 128), lambda i: (i, 0)),
            pl.BlockSpec((_BLK_ROWS, 128), lambda i: (i, 0)),
            pl.BlockSpec((_BLK_ROWS, 128), lambda i: (i, 0)),
        ],
        out_shape=out_shape,
    )(coords3, code, Wc, W1, b1r, W2, b2r)
    return idx.reshape(-1), s01.reshape(-1), s23.reshape(-1)


def _wid():
    return lax.axis_index("s") * 2 + lax.axis_index("c")


def _iota16():
    return lax.iota(jnp.int32, 16)


def _scan_count_base():
    cnt0, _ = plsc.scan_count(jnp.zeros((16,), jnp.int32))
    return jnp.min(cnt0)


_SC_MESH = functools.partial(
    plsc.VectorSubcoreMesh, core_axis_name="c", subcore_axis_name="s")
_SC_PARAMS = pltpu.CompilerParams(needs_layout_passes=False)


def _route_body(idx_hbm, s01_hbm, s23_hbm,
                rloc_hbm, r01_hbm, r23_hbm, hist_hbm,
                idx_v, s01_v, s23_v, hist_v, lbase_v,
                loc_v, o01_v, o23_v, sem):
    w = _wid()
    off = _scan_count_base()
    cp1 = pltpu.async_copy(idx_hbm.at[pl.ds(w * _SPW, _SPW)], idx_v, sem)
    cp2 = pltpu.async_copy(s01_hbm.at[pl.ds(w * _SPW, _SPW)], s01_v, sem)
    cp3 = pltpu.async_copy(s23_hbm.at[pl.ds(w * _SPW, _SPW)], s23_v, sem)
    cp1.wait()
    cp2.wait()
    cp3.wait()
    for c in range(4):
        hist_v[pl.ds(c * 16, 16)] = jnp.zeros((16,), jnp.int32)

    def hbody(i, carry):
        v = idx_v[pl.ds(i * 16, 16)]
        digit = jax.lax.shift_right_logical(v, _BSHIFT)
        cnt, last = plsc.scan_count(digit)
        g = plsc.load_gather(hist_v, [digit])
        plsc.store_scatter(hist_v, [digit], g + cnt - off + 1, mask=last)
        return carry

    lax.fori_loop(0, _SPW // 16, hbody, 0)
    cph = pltpu.async_copy(hist_v, hist_hbm.at[w], sem)

    carry = jnp.zeros((), jnp.int32)
    for c in range(4):
        t = hist_v[pl.ds(c * 16, 16)]
        lbase_v[pl.ds(c * 16, 16)] = plsc.cumsum(t) - t + carry
        carry = carry + jnp.sum(t)

    def pbody(i, carry):
        v = idx_v[pl.ds(i * 16, 16)]
        digit = jax.lax.shift_right_logical(v, _BSHIFT)
        cnt, last = plsc.scan_count(digit)
        rank = cnt - off
        g = plsc.load_gather(lbase_v, [digit])
        dest = g + rank
        plsc.store_scatter(lbase_v, [digit], dest + 1, mask=last)
        plsc.store_scatter(loc_v, [dest], v & jnp.int32(_BCELLS - 1))
        plsc.store_scatter(o01_v, [dest],
                           plsc.bitcast(s01_v[pl.ds(i * 16, 16)], jnp.int32))
        plsc.store_scatter(o23_v, [dest],
                           plsc.bitcast(s23_v[pl.ds(i * 16, 16)], jnp.int32))
        return carry

    lax.fori_loop(0, _SPW // 16, pbody, 0)

    co1 = pltpu.async_copy(loc_v, rloc_hbm.at[pl.ds(w * _SPW, _SPW)], sem)
    co2 = pltpu.async_copy(o01_v, r01_hbm.at[pl.ds(w * _SPW, _SPW)], sem)
    co3 = pltpu.async_copy(o23_v, r23_hbm.at[pl.ds(w * _SPW, _SPW)], sem)
    cph.wait()
    co1.wait()
    co2.wait()
    co3.wait()


def _route_stage(idx, s01, s23):
    k = pl.kernel(
        _route_body,
        out_type=[
            jax.ShapeDtypeStruct((_N + _PAD,), jnp.int32),
            jax.ShapeDtypeStruct((_N + _PAD,), jnp.int32),
            jax.ShapeDtypeStruct((_N + _PAD,), jnp.int32),
            jax.ShapeDtypeStruct((_NW, _NB), jnp.int32),
        ],
        mesh=_SC_MESH(),
        compiler_params=_SC_PARAMS,
        scratch_types=[
            pltpu.VMEM((_SPW,), jnp.int32),
            pltpu.VMEM((_SPW,), jnp.uint32),
            pltpu.VMEM((_SPW,), jnp.uint32),
            pltpu.VMEM((_NB,), jnp.int32),
            pltpu.VMEM((_NB,), jnp.int32),
            pltpu.VMEM((_SPW,), jnp.int32),
            pltpu.VMEM((_SPW,), jnp.int32),
            pltpu.VMEM((_SPW,), jnp.int32),
            pltpu.SemaphoreType.DMA,
        ],
    )
    return k(idx, s01, s23)


def _extract(row_ref, wp, col):
    iot = _iota16()
    acc = jnp.zeros((), jnp.int32)
    for c in range(4):
        t = row_ref[wp, pl.ds(c * 16, 16)]
        acc = acc + jnp.sum(jnp.where(iot + c * 16 == col, t, 0))
    return acc


def _apply_body(dg_hbm, rloc_hbm, r01_hbm, r23_hbm, hist_hbm,
                out_hbm, part_hbm,
                d0_v, d1_v, tmp_v, sl_v, s01_v, s23_v, hist_v,
                meta_s, acc_v, sem, sem_d, sem_o0, sem_o1):
    w = _wid()
    iot = _iota16()
    pltpu.async_copy(hist_hbm, hist_v, sem_d).wait()

    def packrow(wp, carry):
        c0 = jnp.zeros((), jnp.int32)
        for c in range(4):
            t = hist_v[wp, pl.ds(c * 16, 16)]
            excl = plsc.cumsum(t) - t + c0
            c0 = c0 + jnp.sum(t)
            hist_v[wp, pl.ds(c * 16, 16)] = (excl << 13) | t
        return carry

    lax.fori_loop(0, _NW, packrow, 0)

    def fill0(i):
        for c in range(8):
            tmp_v[pl.ds((i * 8 + c) * 16, 16)] = jnp.zeros((16,), jnp.int32)

    plsc.parallel_loop(0, _BCELLS // 128, 1, unroll=2)(fill0)

    acc = jnp.zeros((16,), jnp.float32)
    pending_out = [None, None]
    for h in range(2):
        b = w * 2 + h

        def stage(wp, carry):
            packed = _extract(hist_v, wp, b)
            glen = packed & jnp.int32(8191)
            gstart = wp * _SPW + (packed >> 13)
            astart = pl.multiple_of(gstart & jnp.int32(~7), 8)
            meta_s[2 * wp] = glen
            meta_s[2 * wp + 1] = gstart - astart
            for k in range(_SLOT // 128):
                src = pl.ds(astart + k * 128, 128)
                dst = pl.ds(wp * _SLOT + k * 128, 128)
                pltpu.async_copy(rloc_hbm.at[src], sl_v.at[dst], sem)
                pltpu.async_copy(r01_hbm.at[src], s01_v.at[dst], sem)
                pltpu.async_copy(r23_hbm.at[src], s23_v.at[dst], sem)
            return carry

        lax.fori_loop(0, _NW, stage, 0)

        def drain(i, carry):
            pltpu.make_async_copy(
                rloc_hbm.at[pl.ds(0, 128)], sl_v.at[pl.ds(0, 128)],
                sem).wait()
            return carry

        lax.fori_loop(0, _NW * (_SLOT // 128) * 3, drain, 0)

        def scatter_pass(mode):
            def segs(wp, carry):
                shift = meta_s[2 * wp + 1]
                endp = shift + meta_s[2 * wp]

                def seg(vi, c2):
                    pos = vi * 16 + iot
                    m = (pos >= shift) & (pos < endp)
                    sl = pl.ds(wp * _SLOT + vi * 16, 16)
                    loc = sl_v[sl] & jnp.int32(_BCELLS - 1)
                    if mode == 0:
                        word = s01_v[sl]
                    elif mode == 1:
                        word = s23_v[sl]
                    else:
                        word = jnp.zeros((16,), jnp.int32)
                    plsc.store_scatter(tmp_v, [loc], word, mask=m)
                    return c2

                lax.fori_loop(0, (endp + 15) // 16, seg, 0)
                return carry

            lax.fori_loop(0, _NW, segs, 0)

        for s in range(_SCENES):
            if pending_out[s % 2] is not None:
                pending_out[s % 2].wait()
                pending_out[s % 2] = None
            dbuf = d0_v if s % 2 == 0 else d1_v
            cpd = pltpu.async_copy(
                dg_hbm.at[s, pl.ds(b * _BCELLS, _BCELLS)], dbuf, sem_d)
            if s % 2 == 0:
                scatter_pass(s // 2)
            cpd.wait()

            def merge(i, a, s=s, dbuf=dbuf):
                for c in range(8):
                    slc = pl.ds((i * 8 + c) * 16, 16)
                    d = dbuf[slc]
                    word = tmp_v[slc]
                    if s % 2 == 0:
                        bits = jax.lax.shift_left(word, 16)
                    else:
                        bits = word & jnp.int32(-65536)
                    t = plsc.bitcast(bits, jnp.float32)
                    o = jnp.where(bits != 0,
                                  jnp.maximum(d * jnp.float32(_DECAY), t), d)
                    dbuf[slc] = o
                    a = a + o
                return a

            acc = plsc.parallel_loop(
                0, _BCELLS // 128, 1, unroll=2, carry=acc)(merge)
            pending_out[s % 2] = pltpu.async_copy(
                dbuf, out_hbm.at[s, pl.ds(b * _BCELLS, _BCELLS)],
                sem_o0 if s % 2 == 0 else sem_o1)
        if h == 0:
            scatter_pass(2)
    for cp in pending_out:
        if cp is not None:
            cp.wait()
    acc_v[...] = acc
    pltpu.async_copy(acc_v, part_hbm.at[w], sem_d).wait()


def _apply_stage(density_grid, rloc, r01, r23, hist):
    k = pl.kernel(
        _apply_body,
        out_type=[
            jax.ShapeDtypeStruct((_SCENES, _CELLS), jnp.float32),
            jax.ShapeDtypeStruct((_NW, 16), jnp.float32),
        ],
        mesh=_SC_MESH(),
        compiler_params=_SC_PARAMS,
        scratch_types=[
            pltpu.VMEM((_BCELLS,), jnp.float32),
            pltpu.VMEM((_BCELLS,), jnp.float32),
            pltpu.VMEM((_BCELLS,), jnp.int32),
            pltpu.VMEM((_NW * _SLOT,), jnp.int32),
            pltpu.VMEM((_NW * _SLOT,), jnp.int32),
            pltpu.VMEM((_NW * _SLOT,), jnp.int32),
            pltpu.VMEM((_NW, _NB), jnp.int32),
            pltpu.SMEM((2 * _NW,), jnp.int32),
            pltpu.VMEM((16,), jnp.float32),
            pltpu.SemaphoreType.DMA,
            pltpu.SemaphoreType.DMA,
            pltpu.SemaphoreType.DMA,
            pltpu.SemaphoreType.DMA,
        ],
    )
    return k(density_grid, rloc, r01, r23, hist)


def kernel(density_grid, code, W1, b1, Wc, W2, b2, coords):
    idx, s01, s23 = _sigma_stage(coords, code, W1, b1, Wc, W2, b2)
    rloc, r01, r23, hist = _route_stage(idx, s01, s23)
    new_grid, partials = _apply_stage(density_grid, rloc, r01, r23, hist)
    mean_density = jnp.sum(partials) / jnp.float32(_SCENES * _CELLS)
    return new_grid, mean_density


# speedup vs baseline: 1.0288x; 1.0288x over previous
import functools

import jax
import jax.numpy as jnp
from jax import lax
from jax.experimental import pallas as pl
from jax.experimental.pallas import tpu as pltpu
from jax.experimental.pallas import tpu_sc as plsc

_GRID = 128
_CELLS = _GRID ** 3
_SCENES = 4
_HIDDEN = 16
_N = 262144
_ROWS = _N // 128
_BLK_ROWS = 256
_DECAY = 0.9

_NW = 32
_SPW = _N // _NW
_NB = 64
_BCELLS = _CELLS // _NB
_BSHIFT = 15
_PAD = 1024
_SLOT = 256


def _part1by2(x):
    x = x & jnp.uint32(0x3FF)
    x = (x | (x << 16)) & jnp.uint32(0x30000FF)
    x = (x | (x << 8)) & jnp.uint32(0x300F00F)
    x = (x | (x << 4)) & jnp.uint32(0x30C30C3)
    x = (x | (x << 2)) & jnp.uint32(0x9249249)
    return x


def _sigma_body(coords_ref, code_ref, Wc_ref, w1_ref, b1_ref, w2_ref, b2_ref,
                idx_ref, s01_ref, s23_ref):
    cx = coords_ref[0]
    cy = coords_ref[1]
    cz = coords_ref[2]
    mx = _part1by2(cx.astype(jnp.uint32))
    my = _part1by2(cy.astype(jnp.uint32))
    mz = _part1by2(cz.astype(jnp.uint32))
    idx_ref[...] = (mx | (my << 1) | (mz << 2)).astype(jnp.int32)
    scale = jnp.float32(2.0 / _GRID)
    half = jnp.float32((_GRID - 1) / 2.0)
    fx = (cx.astype(jnp.float32) - half) * scale
    fy = (cy.astype(jnp.float32) - half) * scale
    fz = (cz.astype(jnp.float32) - half) * scale
    cw = jnp.dot(code_ref[...], Wc_ref[...],
                 preferred_element_type=jnp.float32) + b1_ref[...]
    w1 = w1_ref[...]
    w2 = w2_ref[...]
    accs = [jnp.zeros(fx.shape, jnp.float32) for _ in range(_SCENES)]
    for j in range(_HIDDEN):
        base = fx * w1[0, j] + fy * w1[1, j] + fz * w1[2, j]
        for s in range(_SCENES):
            accs[s] = accs[s] + jnp.maximum(base + cw[s, j], 0.0) * w2[j, 0]
    b2v = b2_ref[0, 0]
    sig = [jax.nn.softplus(a + b2v) for a in accs]
    bits = [jax.lax.bitcast_convert_type(
        s.astype(jnp.bfloat16), jnp.uint16).astype(jnp.uint32) for s in sig]
    s01_ref[...] = bits[0] | (bits[1] << 16)
    s23_ref[...] = bits[2] | (bits[3] << 16)


def _sigma_stage(coords, code, W1, b1, Wc, W2, b2):
    coords3 = coords.T.reshape(3, _ROWS, 128)
    grid = (_ROWS // _BLK_ROWS,)
    full = lambda a: pl.BlockSpec(a.shape, lambda i: tuple(0 for _ in a.shape))
    b1r = b1.reshape(1, _HIDDEN)
    b2r = b2.reshape(1, 1)
    out_shape = [
        jax.ShapeDtypeStruct((_ROWS, 128), jnp.int32),
        jax.ShapeDtypeStruct((_ROWS, 128), jnp.uint32),
        jax.ShapeDtypeStruct((_ROWS, 128), jnp.uint32),
    ]
    idx, s01, s23 = pl.pallas_call(
        _sigma_body,
        grid=grid,
        in_specs=[
            pl.BlockSpec((3, _BLK_ROWS, 128), lambda i: (0, i, 0)),
            full(code), full(Wc), full(W1), full(b1r), full(W2), full(b2r),
        ],
        out_specs=[
            pl.BlockSpec((_BLK_ROWS, 128), lambda i: (i, 0)),
            pl.BlockSpec((_BLK_ROWS, 128), lambda i: (i, 0)),
            pl.BlockSpec((_BLK_ROWS, 128), lambda i: (i, 0)),
        ],
        out_shape=out_shape,
    )(coords3, code, Wc, W1, b1r, W2, b2r)
    return idx.reshape(-1), s01.reshape(-1), s23.reshape(-1)


def _wid():
    return lax.axis_index("s") * 2 + lax.axis_index("c")


def _iota16():
    return lax.iota(jnp.int32, 16)


def _scan_count_base():
    cnt0, _ = plsc.scan_count(jnp.zeros((16,), jnp.int32))
    return jnp.min(cnt0)


_SC_MESH = functools.partial(
    plsc.VectorSubcoreMesh, core_axis_name="c", subcore_axis_name="s")
_SC_PARAMS = pltpu.CompilerParams(needs_layout_passes=False)


def _route_body(idx_hbm, s01_hbm, s23_hbm,
                rloc_hbm, r01_hbm, r23_hbm, hist_hbm,
                idx_v, s01_v, s23_v, hist_v, lbase_v,
                loc_v, o01_v, o23_v, sem):
    w = _wid()
    off = _scan_count_base()
    cp1 = pltpu.async_copy(idx_hbm.at[pl.ds(w * _SPW, _SPW)], idx_v, sem)
    cp2 = pltpu.async_copy(s01_hbm.at[pl.ds(w * _SPW, _SPW)], s01_v, sem)
    cp3 = pltpu.async_copy(s23_hbm.at[pl.ds(w * _SPW, _SPW)], s23_v, sem)
    cp1.wait()
    cp2.wait()
    cp3.wait()
    for c in range(4):
        hist_v[pl.ds(c * 16, 16)] = jnp.zeros((16,), jnp.int32)

    def hbody(i, carry):
        v = idx_v[pl.ds(i * 16, 16)]
        digit = jax.lax.shift_right_logical(v, _BSHIFT)
        cnt, last = plsc.scan_count(digit)
        g = plsc.load_gather(hist_v, [digit])
        plsc.store_scatter(hist_v, [digit], g + cnt - off + 1, mask=last)
        return carry

    lax.fori_loop(0, _SPW // 16, hbody, 0)
    cph = pltpu.async_copy(hist_v, hist_hbm.at[w], sem)

    carry = jnp.zeros((), jnp.int32)
    for c in range(4):
        t = hist_v[pl.ds(c * 16, 16)]
        lbase_v[pl.ds(c * 16, 16)] = plsc.cumsum(t) - t + carry
        carry = carry + jnp.sum(t)

    def pbody(i, carry):
        v = idx_v[pl.ds(i * 16, 16)]
        digit = jax.lax.shift_right_logical(v, _BSHIFT)
        cnt, last = plsc.scan_count(digit)
        rank = cnt - off
        g = plsc.load_gather(lbase_v, [digit])
        dest = g + rank
        plsc.store_scatter(lbase_v, [digit], dest + 1, mask=last)
        plsc.store_scatter(loc_v, [dest], v & jnp.int32(_BCELLS - 1))
        plsc.store_scatter(o01_v, [dest],
                           plsc.bitcast(s01_v[pl.ds(i * 16, 16)], jnp.int32))
        plsc.store_scatter(o23_v, [dest],
                           plsc.bitcast(s23_v[pl.ds(i * 16, 16)], jnp.int32))
        return carry

    lax.fori_loop(0, _SPW // 16, pbody, 0)

    co1 = pltpu.async_copy(loc_v, rloc_hbm.at[pl.ds(w * _SPW, _SPW)], sem)
    co2 = pltpu.async_copy(o01_v, r01_hbm.at[pl.ds(w * _SPW, _SPW)], sem)
    co3 = pltpu.async_copy(o23_v, r23_hbm.at[pl.ds(w * _SPW, _SPW)], sem)
    cph.wait()
    co1.wait()
    co2.wait()
    co3.wait()


def _route_stage(idx, s01, s23):
    k = pl.kernel(
        _route_body,
        out_type=[
            jax.ShapeDtypeStruct((_N + _PAD,), jnp.int32),
            jax.ShapeDtypeStruct((_N + _PAD,), jnp.int32),
            jax.ShapeDtypeStruct((_N + _PAD,), jnp.int32),
            jax.ShapeDtypeStruct((_NW, _NB), jnp.int32),
        ],
        mesh=_SC_MESH(),
        compiler_params=_SC_PARAMS,
        scratch_types=[
            pltpu.VMEM((_SPW,), jnp.int32),
            pltpu.VMEM((_SPW,), jnp.uint32),
            pltpu.VMEM((_SPW,), jnp.uint32),
            pltpu.VMEM((_NB,), jnp.int32),
            pltpu.VMEM((_NB,), jnp.int32),
            pltpu.VMEM((_SPW,), jnp.int32),
            pltpu.VMEM((_SPW,), jnp.int32),
            pltpu.VMEM((_SPW,), jnp.int32),
            pltpu.SemaphoreType.DMA,
        ],
    )
    return k(idx, s01, s23)


def _extract(row_ref, wp, col):
    iot = _iota16()
    acc = jnp.zeros((), jnp.int32)
    for c in range(4):
        t = row_ref[wp, pl.ds(c * 16, 16)]
        acc = acc + jnp.sum(jnp.where(iot + c * 16 == col, t, 0))
    return acc


def _apply_body(dg_hbm, rloc_hbm, r01_hbm, r23_hbm, hist_hbm,
                out_hbm, part_hbm,
                d0_v, d1_v, tmp_v, sl_v, s01_v, s23_v, hist_v,
                meta_s, acc_v, sem, sem_d, sem_o0, sem_o1):
    w = _wid()
    iot = _iota16()
    pltpu.async_copy(hist_hbm, hist_v, sem_d).wait()

    def packrow(wp, carry):
        c0 = jnp.zeros((), jnp.int32)
        for c in range(4):
            t = hist_v[wp, pl.ds(c * 16, 16)]
            excl = plsc.cumsum(t) - t + c0
            c0 = c0 + jnp.sum(t)
            hist_v[wp, pl.ds(c * 16, 16)] = (excl << 13) | t
        return carry

    lax.fori_loop(0, _NW, packrow, 0)

    def fill0(i, carry):
        for c in range(8):
            tmp_v[pl.ds((i * 8 + c) * 16, 16)] = jnp.zeros((16,), jnp.int32)
        return carry

    lax.fori_loop(0, _BCELLS // 128, fill0, 0)

    acc = jnp.zeros((16,), jnp.float32)
    pending_out = [None, None]
    for h in range(2):
        b = w * 2 + h

        def stage(wp, carry):
            packed = _extract(hist_v, wp, b)
            glen = packed & jnp.int32(8191)
            gstart = wp * _SPW + (packed >> 13)
            astart = pl.multiple_of(gstart & jnp.int32(~7), 8)
            meta_s[2 * wp] = glen
            meta_s[2 * wp + 1] = gstart - astart
            for k in range(_SLOT // 128):
                src = pl.ds(astart + k * 128, 128)
                dst = pl.ds(wp * _SLOT + k * 128, 128)
                pltpu.async_copy(rloc_hbm.at[src], sl_v.at[dst], sem)
                pltpu.async_copy(r01_hbm.at[src], s01_v.at[dst], sem)
                pltpu.async_copy(r23_hbm.at[src], s23_v.at[dst], sem)
            return carry

        lax.fori_loop(0, _NW, stage, 0)

        def drain(i, carry):
            pltpu.make_async_copy(
                rloc_hbm.at[pl.ds(0, 128)], sl_v.at[pl.ds(0, 128)],
                sem).wait()
            return carry

        lax.fori_loop(0, _NW * (_SLOT // 128) * 3, drain, 0)

        def scatter_pass(mode):
            def segs(wp, carry):
                shift = meta_s[2 * wp + 1]
                endp = shift + meta_s[2 * wp]

                def seg(vi, c2):
                    pos = vi * 16 + iot
                    m = (pos >= shift) & (pos < endp)
                    sl = pl.ds(wp * _SLOT + vi * 16, 16)
                    loc = sl_v[sl] & jnp.int32(_BCELLS - 1)
                    if mode == 0:
                        word = s01_v[sl]
                    elif mode == 1:
                        word = s23_v[sl]
                    else:
                        word = jnp.zeros((16,), jnp.int32)
                    plsc.store_scatter(tmp_v, [loc], word, mask=m)
                    return c2

                lax.fori_loop(0, (endp + 15) // 16, seg, 0)
                return carry

            lax.fori_loop(0, _NW, segs, 0)

        for pair in range(2):
            s0, s1 = 2 * pair, 2 * pair + 1
            if pending_out[0] is not None:
                pending_out[0].wait()
                pending_out[1].wait()
                pending_out = [None, None]
            cpd0 = pltpu.async_copy(
                dg_hbm.at[s0, pl.ds(b * _BCELLS, _BCELLS)], d0_v, sem_d)
            cpd1 = pltpu.async_copy(
                dg_hbm.at[s1, pl.ds(b * _BCELLS, _BCELLS)], d1_v, sem_d)
            scatter_pass(pair)
            cpd0.wait()
            cpd1.wait()

            def merge(i, a):
                for c in range(8):
                    slc = pl.ds((i * 8 + c) * 16, 16)
                    word = tmp_v[slc]
                    d0 = d0_v[slc]
                    d1 = d1_v[slc]
                    bits0 = jax.lax.shift_left(word, 16)
                    bits1 = word & jnp.int32(-65536)
                    t0 = plsc.bitcast(bits0, jnp.float32)
                    t1 = plsc.bitcast(bits1, jnp.float32)
                    o0 = jnp.where(bits0 != 0,
                                   jnp.maximum(d0 * jnp.float32(_DECAY), t0),
                                   d0)
                    o1 = jnp.where(bits1 != 0,
                                   jnp.maximum(d1 * jnp.float32(_DECAY), t1),
                                   d1)
                    d0_v[slc] = o0
                    d1_v[slc] = o1
                    a = a + o0 + o1
                return a

            acc = lax.fori_loop(0, _BCELLS // 128, merge, acc)
            pending_out = [
                pltpu.async_copy(
                    d0_v, out_hbm.at[s0, pl.ds(b * _BCELLS, _BCELLS)],
                    sem_o0),
                pltpu.async_copy(
                    d1_v, out_hbm.at[s1, pl.ds(b * _BCELLS, _BCELLS)],
                    sem_o1),
            ]
        if h == 0:
            scatter_pass(2)
    for cp in pending_out:
        if cp is not None:
            cp.wait()
    acc_v[...] = acc
    pltpu.async_copy(acc_v, part_hbm.at[w], sem_d).wait()


def _apply_stage(density_grid, rloc, r01, r23, hist):
    k = pl.kernel(
        _apply_body,
        out_type=[
            jax.ShapeDtypeStruct((_SCENES, _CELLS), jnp.float32),
            jax.ShapeDtypeStruct((_NW, 16), jnp.float32),
        ],
        mesh=_SC_MESH(),
        compiler_params=_SC_PARAMS,
        scratch_types=[
            pltpu.VMEM((_BCELLS,), jnp.float32),
            pltpu.VMEM((_BCELLS,), jnp.float32),
            pltpu.VMEM((_BCELLS,), jnp.int32),
            pltpu.VMEM((_NW * _SLOT,), jnp.int32),
            pltpu.VMEM((_NW * _SLOT,), jnp.int32),
            pltpu.VMEM((_NW * _SLOT,), jnp.int32),
            pltpu.VMEM((_NW, _NB), jnp.int32),
            pltpu.SMEM((2 * _NW,), jnp.int32),
            pltpu.VMEM((16,), jnp.float32),
            pltpu.SemaphoreType.DMA,
            pltpu.SemaphoreType.DMA,
            pltpu.SemaphoreType.DMA,
            pltpu.SemaphoreType.DMA,
        ],
    )
    return k(density_grid, rloc, r01, r23, hist)


def kernel(density_grid, code, W1, b1, Wc, W2, b2, coords):
    idx, s01, s23 = _sigma_stage(coords, code, W1, b1, Wc, W2, b2)
    rloc, r01, r23, hist = _route_stage(idx, s01, s23)
    new_grid, partials = _apply_stage(density_grid, rloc, r01, r23, hist)
    mean_density = jnp.sum(partials) / jnp.float32(_SCENES * _CELLS)
    return new_grid, mean_density


# speedup vs baseline: 1.0394x; 1.0103x over previous
import functools

import jax
import jax.numpy as jnp
from jax import lax
from jax.experimental import pallas as pl
from jax.experimental.pallas import tpu as pltpu
from jax.experimental.pallas import tpu_sc as plsc

_GRID = 128
_CELLS = _GRID ** 3
_SCENES = 4
_HIDDEN = 16
_N = 262144
_ROWS = _N // 128
_BLK_ROWS = 1024
_DECAY = 0.9

_NW = 32
_SPW = _N // _NW
_NB = 64
_BCELLS = _CELLS // _NB
_BSHIFT = 15
_PAD = 1024
_SLOT = 256


def _part1by2(x):
    x = x & jnp.uint32(0x3FF)
    x = (x | (x << 16)) & jnp.uint32(0x30000FF)
    x = (x | (x << 8)) & jnp.uint32(0x300F00F)
    x = (x | (x << 4)) & jnp.uint32(0x30C30C3)
    x = (x | (x << 2)) & jnp.uint32(0x9249249)
    return x


def _sigma_body(coords_ref, code_ref, Wc_ref, w1_ref, b1_ref, w2_ref, b2_ref,
                idx_ref, s01_ref, s23_ref):
    cx = coords_ref[0]
    cy = coords_ref[1]
    cz = coords_ref[2]
    mx = _part1by2(cx.astype(jnp.uint32))
    my = _part1by2(cy.astype(jnp.uint32))
    mz = _part1by2(cz.astype(jnp.uint32))
    idx_ref[...] = (mx | (my << 1) | (mz << 2)).astype(jnp.int32)
    scale = jnp.float32(2.0 / _GRID)
    half = jnp.float32((_GRID - 1) / 2.0)
    fx = (cx.astype(jnp.float32) - half) * scale
    fy = (cy.astype(jnp.float32) - half) * scale
    fz = (cz.astype(jnp.float32) - half) * scale
    cw = jnp.dot(code_ref[...], Wc_ref[...],
                 preferred_element_type=jnp.float32) + b1_ref[...]
    w1 = w1_ref[...]
    w2 = w2_ref[...]
    accs = [jnp.zeros(fx.shape, jnp.float32) for _ in range(_SCENES)]
    for j in range(_HIDDEN):
        base = fx * w1[0, j] + fy * w1[1, j] + fz * w1[2, j]
        for s in range(_SCENES):
            accs[s] = accs[s] + jnp.maximum(base + cw[s, j], 0.0) * w2[j, 0]
    b2v = b2_ref[0, 0]
    sig = [jax.nn.softplus(a + b2v) for a in accs]
    bits = [jax.lax.bitcast_convert_type(
        s.astype(jnp.bfloat16), jnp.uint16).astype(jnp.uint32) for s in sig]
    s01_ref[...] = bits[0] | (bits[1] << 16)
    s23_ref[...] = bits[2] | (bits[3] << 16)


def _sigma_stage(coords, code, W1, b1, Wc, W2, b2):
    coords3 = coords.T.reshape(3, _ROWS, 128)
    grid = (_ROWS // _BLK_ROWS,)
    full = lambda a: pl.BlockSpec(a.shape, lambda i: tuple(0 for _ in a.shape))
    b1r = b1.reshape(1, _HIDDEN)
    b2r = b2.reshape(1, 1)
    out_shape = [
        jax.ShapeDtypeStruct((_ROWS, 128), jnp.int32),
        jax.ShapeDtypeStruct((_ROWS, 128), jnp.uint32),
        jax.ShapeDtypeStruct((_ROWS, 128), jnp.uint32),
    ]
    idx, s01, s23 = pl.pallas_call(
        _sigma_body,
        grid=grid,
        in_specs=[
            pl.BlockSpec((3, _BLK_ROWS, 128), lambda i: (0, i, 0)),
            full(code), full(Wc), full(W1), full(b1r), full(W2), full(b2r),
        ],
        out_specs=[
            pl.BlockSpec((_BLK_ROWS, 128), lambda i: (i, 0)),
            pl.BlockSpec((_BLK_ROWS, 128), lambda i: (i, 0)),
            pl.BlockSpec((_BLK_ROWS, 128), lambda i: (i, 0)),
        ],
        out_shape=out_shape,
    )(coords3, code, Wc, W1, b1r, W2, b2r)
    return idx.reshape(-1), s01.reshape(-1), s23.reshape(-1)


def _wid():
    return lax.axis_index("s") * 2 + lax.axis_index("c")


def _iota16():
    return lax.iota(jnp.int32, 16)


def _scan_count_base():
    cnt0, _ = plsc.scan_count(jnp.zeros((16,), jnp.int32))
    return jnp.min(cnt0)


_SC_MESH = functools.partial(
    plsc.VectorSubcoreMesh, core_axis_name="c", subcore_axis_name="s")
_SC_PARAMS = pltpu.CompilerParams(needs_layout_passes=False)


def _route_body(idx_hbm, s01_hbm, s23_hbm,
                rloc_hbm, r01_hbm, r23_hbm, hist_hbm,
                idx_v, s01_v, s23_v, hist_v, lbase_v, rank_v,
                loc_v, o01_v, o23_v, sem):
    w = _wid()
    off = _scan_count_base()
    cp1 = pltpu.async_copy(idx_hbm.at[pl.ds(w * _SPW, _SPW)], idx_v, sem)
    cp2 = pltpu.async_copy(s01_hbm.at[pl.ds(w * _SPW, _SPW)], s01_v, sem)
    cp3 = pltpu.async_copy(s23_hbm.at[pl.ds(w * _SPW, _SPW)], s23_v, sem)
    cp1.wait()
    cp2.wait()
    cp3.wait()
    for c in range(4):
        hist_v[pl.ds(c * 16, 16)] = jnp.zeros((16,), jnp.int32)

    def hbody(i, carry):
        v = idx_v[pl.ds(i * 16, 16)]
        digit = jax.lax.shift_right_logical(v, _BSHIFT)
        cnt, last = plsc.scan_count(digit)
        g = plsc.load_gather(hist_v, [digit])
        crank = g + cnt - off
        rank_v[pl.ds(i * 16, 16)] = crank
        plsc.store_scatter(hist_v, [digit], crank + 1, mask=last)
        return carry

    lax.fori_loop(0, _SPW // 16, hbody, 0)
    cph = pltpu.async_copy(hist_v, hist_hbm.at[w], sem)

    carry = jnp.zeros((), jnp.int32)
    for c in range(4):
        t = hist_v[pl.ds(c * 16, 16)]
        lbase_v[pl.ds(c * 16, 16)] = plsc.cumsum(t) - t + carry
        carry = carry + jnp.sum(t)

    def pbody(i):
        v = idx_v[pl.ds(i * 16, 16)]
        digit = jax.lax.shift_right_logical(v, _BSHIFT)
        g = plsc.load_gather(lbase_v, [digit])
        dest = g + rank_v[pl.ds(i * 16, 16)]
        plsc.store_scatter(loc_v, [dest], v & jnp.int32(_BCELLS - 1))
        plsc.store_scatter(o01_v, [dest],
                           plsc.bitcast(s01_v[pl.ds(i * 16, 16)], jnp.int32))
        plsc.store_scatter(o23_v, [dest],
                           plsc.bitcast(s23_v[pl.ds(i * 16, 16)], jnp.int32))

    plsc.parallel_loop(0, _SPW // 16, 1, unroll=2)(pbody)

    co1 = pltpu.async_copy(loc_v, rloc_hbm.at[pl.ds(w * _SPW, _SPW)], sem)
    co2 = pltpu.async_copy(o01_v, r01_hbm.at[pl.ds(w * _SPW, _SPW)], sem)
    co3 = pltpu.async_copy(o23_v, r23_hbm.at[pl.ds(w * _SPW, _SPW)], sem)
    cph.wait()
    co1.wait()
    co2.wait()
    co3.wait()


def _route_stage(idx, s01, s23):
    k = pl.kernel(
        _route_body,
        out_type=[
            jax.ShapeDtypeStruct((_N + _PAD,), jnp.int32),
            jax.ShapeDtypeStruct((_N + _PAD,), jnp.int32),
            jax.ShapeDtypeStruct((_N + _PAD,), jnp.int32),
            jax.ShapeDtypeStruct((_NW, _NB), jnp.int32),
        ],
        mesh=_SC_MESH(),
        compiler_params=_SC_PARAMS,
        scratch_types=[
            pltpu.VMEM((_SPW,), jnp.int32),
            pltpu.VMEM((_SPW,), jnp.uint32),
            pltpu.VMEM((_SPW,), jnp.uint32),
            pltpu.VMEM((_NB,), jnp.int32),
            pltpu.VMEM((_NB,), jnp.int32),
            pltpu.VMEM((_SPW,), jnp.int32),
            pltpu.VMEM((_SPW,), jnp.int32),
            pltpu.VMEM((_SPW,), jnp.int32),
            pltpu.VMEM((_SPW,), jnp.int32),
            pltpu.SemaphoreType.DMA,
        ],
    )
    return k(idx, s01, s23)


def _extract(row_ref, wp, col):
    iot = _iota16()
    acc = jnp.zeros((), jnp.int32)
    for c in range(4):
        t = row_ref[wp, pl.ds(c * 16, 16)]
        acc = acc + jnp.sum(jnp.where(iot + c * 16 == col, t, 0))
    return acc


def _apply_body(dg_hbm, rloc_hbm, r01_hbm, r23_hbm, hist_hbm,
                out_hbm, part_hbm,
                d0_v, d1_v, tmp_v, sl_v, s01_v, s23_v, hist_v,
                meta_s, acc_v, sem, sem_d, sem_o0, sem_o1):
    w = _wid()
    iot = _iota16()
    pltpu.async_copy(hist_hbm, hist_v, sem_d).wait()

    def packrow(wp, carry):
        c0 = jnp.zeros((), jnp.int32)
        for c in range(4):
            t = hist_v[wp, pl.ds(c * 16, 16)]
            excl = plsc.cumsum(t) - t + c0
            c0 = c0 + jnp.sum(t)
            hist_v[wp, pl.ds(c * 16, 16)] = (excl << 13) | t
        return carry

    lax.fori_loop(0, _NW, packrow, 0)

    def fill0(i, carry):
        for c in range(8):
            tmp_v[pl.ds((i * 8 + c) * 16, 16)] = jnp.zeros((16,), jnp.int32)
        return carry

    lax.fori_loop(0, _BCELLS // 128, fill0, 0)

    acc = jnp.zeros((16,), jnp.float32)
    pending_out = [None, None]
    for h in range(2):
        b = w * 2 + h

        def stage(wp, carry):
            packed = _extract(hist_v, wp, b)
            glen = packed & jnp.int32(8191)
            gstart = wp * _SPW + (packed >> 13)
            astart = pl.multiple_of(gstart & jnp.int32(~7), 8)
            meta_s[2 * wp] = glen
            meta_s[2 * wp + 1] = gstart - astart
            for k in range(_SLOT // 128):
                src = pl.ds(astart + k * 128, 128)
                dst = pl.ds(wp * _SLOT + k * 128, 128)
                pltpu.async_copy(rloc_hbm.at[src], sl_v.at[dst], sem)
                pltpu.async_copy(r01_hbm.at[src], s01_v.at[dst], sem)
                pltpu.async_copy(r23_hbm.at[src], s23_v.at[dst], sem)
            return carry

        lax.fori_loop(0, _NW, stage, 0)

        def drain(i, carry):
            pltpu.make_async_copy(
                rloc_hbm.at[pl.ds(0, 128)], sl_v.at[pl.ds(0, 128)],
                sem).wait()
            return carry

        lax.fori_loop(0, _NW * (_SLOT // 128) * 3, drain, 0)

        def scatter_pass(mode):
            def segs(wp, carry):
                shift = meta_s[2 * wp + 1]
                endp = shift + meta_s[2 * wp]

                def seg(vi, c2):
                    pos = vi * 16 + iot
                    m = (pos >= shift) & (pos < endp)
                    sl = pl.ds(wp * _SLOT + vi * 16, 16)
                    loc = sl_v[sl] & jnp.int32(_BCELLS - 1)
                    if mode == 0:
                        word = s01_v[sl]
                    elif mode == 1:
                        word = s23_v[sl]
                    else:
                        word = jnp.zeros((16,), jnp.int32)
                    plsc.store_scatter(tmp_v, [loc], word, mask=m)
                    return c2

                lax.fori_loop(0, (endp + 15) // 16, seg, 0)
                return carry

            lax.fori_loop(0, _NW, segs, 0)

        for pair in range(2):
            s0, s1 = 2 * pair, 2 * pair + 1
            if pending_out[0] is not None:
                pending_out[0].wait()
                pending_out[1].wait()
                pending_out = [None, None]
            cpd0 = pltpu.async_copy(
                dg_hbm.at[s0, pl.ds(b * _BCELLS, _BCELLS)], d0_v, sem_d)
            cpd1 = pltpu.async_copy(
                dg_hbm.at[s1, pl.ds(b * _BCELLS, _BCELLS)], d1_v, sem_d)
            scatter_pass(pair)
            cpd0.wait()
            cpd1.wait()

            def merge(i, a):
                for c in range(8):
                    slc = pl.ds((i * 8 + c) * 16, 16)
                    word = tmp_v[slc]
                    d0 = d0_v[slc]
                    d1 = d1_v[slc]
                    bits0 = jax.lax.shift_left(word, 16)
                    bits1 = word & jnp.int32(-65536)
                    t0 = plsc.bitcast(bits0, jnp.float32)
                    t1 = plsc.bitcast(bits1, jnp.float32)
                    o0 = jnp.where(bits0 != 0,
                                   jnp.maximum(d0 * jnp.float32(_DECAY), t0),
                                   d0)
                    o1 = jnp.where(bits1 != 0,
                                   jnp.maximum(d1 * jnp.float32(_DECAY), t1),
                                   d1)
                    d0_v[slc] = o0
                    d1_v[slc] = o1
                    a = a + o0 + o1
                return a

            acc = lax.fori_loop(0, _BCELLS // 128, merge, acc)
            pending_out = [
                pltpu.async_copy(
                    d0_v, out_hbm.at[s0, pl.ds(b * _BCELLS, _BCELLS)],
                    sem_o0),
                pltpu.async_copy(
                    d1_v, out_hbm.at[s1, pl.ds(b * _BCELLS, _BCELLS)],
                    sem_o1),
            ]
        if h == 0:
            scatter_pass(2)
    for cp in pending_out:
        if cp is not None:
            cp.wait()
    acc_v[...] = acc
    pltpu.async_copy(acc_v, part_hbm.at[w], sem_d).wait()


def _apply_stage(density_grid, rloc, r01, r23, hist):
    k = pl.kernel(
        _apply_body,
        out_type=[
            jax.ShapeDtypeStruct((_SCENES, _CELLS), jnp.float32),
            jax.ShapeDtypeStruct((_NW, 16), jnp.float32),
        ],
        mesh=_SC_MESH(),
        compiler_params=_SC_PARAMS,
        scratch_types=[
            pltpu.VMEM((_BCELLS,), jnp.float32),
            pltpu.VMEM((_BCELLS,), jnp.float32),
            pltpu.VMEM((_BCELLS,), jnp.int32),
            pltpu.VMEM((_NW * _SLOT,), jnp.int32),
            pltpu.VMEM((_NW * _SLOT,), jnp.int32),
            pltpu.VMEM((_NW * _SLOT,), jnp.int32),
            pltpu.VMEM((_NW, _NB), jnp.int32),
            pltpu.SMEM((2 * _NW,), jnp.int32),
            pltpu.VMEM((16,), jnp.float32),
            pltpu.SemaphoreType.DMA,
            pltpu.SemaphoreType.DMA,
            pltpu.SemaphoreType.DMA,
            pltpu.SemaphoreType.DMA,
        ],
    )
    return k(density_grid, rloc, r01, r23, hist)


def kernel(density_grid, code, W1, b1, Wc, W2, b2, coords):
    idx, s01, s23 = _sigma_stage(coords, code, W1, b1, Wc, W2, b2)
    rloc, r01, r23, hist = _route_stage(idx, s01, s23)
    new_grid, partials = _apply_stage(density_grid, rloc, r01, r23, hist)
    mean_density = jnp.sum(partials) / jnp.float32(_SCENES * _CELLS)
    return new_grid, mean_density


# speedup vs baseline: 1.1246x; 1.0820x over previous
import functools

import jax
import jax.numpy as jnp
from jax import lax
from jax.experimental import pallas as pl
from jax.experimental.pallas import tpu as pltpu
from jax.experimental.pallas import tpu_sc as plsc

_GRID = 128
_CELLS = _GRID ** 3
_SCENES = 4
_HIDDEN = 16
_N = 262144
_ROWS = _N // 128
_BLK_ROWS = 256
_DECAY = 0.9

_NW = 32
_SPW = _N // _NW
_NB = 64
_BCELLS = _CELLS // _NB
_BSHIFT = 15
_PAD = 1024
_SLOT = 256


def _part1by2(x):
    x = x & jnp.uint32(0x3FF)
    x = (x | (x << 16)) & jnp.uint32(0x30000FF)
    x = (x | (x << 8)) & jnp.uint32(0x300F00F)
    x = (x | (x << 4)) & jnp.uint32(0x30C30C3)
    x = (x | (x << 2)) & jnp.uint32(0x9249249)
    return x


def _sigma_body(coords_ref, code_ref, Wc_ref, w1_ref, b1_ref, w2_ref, b2_ref,
                idx_ref, s01_ref, s23_ref):
    cx = coords_ref[0]
    cy = coords_ref[1]
    cz = coords_ref[2]
    mx = _part1by2(cx.astype(jnp.uint32))
    my = _part1by2(cy.astype(jnp.uint32))
    mz = _part1by2(cz.astype(jnp.uint32))
    idx_ref[...] = (mx | (my << 1) | (mz << 2)).astype(jnp.int32)
    scale = jnp.float32(2.0 / _GRID)
    half = jnp.float32((_GRID - 1) / 2.0)
    fx = (cx.astype(jnp.float32) - half) * scale
    fy = (cy.astype(jnp.float32) - half) * scale
    fz = (cz.astype(jnp.float32) - half) * scale
    cw = jnp.dot(code_ref[...], Wc_ref[...],
                 preferred_element_type=jnp.float32) + b1_ref[...]
    w1 = w1_ref[...]
    w2 = w2_ref[...]
    accs = [jnp.zeros(fx.shape, jnp.float32) for _ in range(_SCENES)]
    for j in range(_HIDDEN):
        base = fx * w1[0, j] + fy * w1[1, j] + fz * w1[2, j]
        for s in range(_SCENES):
            accs[s] = accs[s] + jnp.maximum(base + cw[s, j], 0.0) * w2[j, 0]
    b2v = b2_ref[0, 0]
    sig = [jax.nn.softplus(a + b2v) for a in accs]
    bits = [jax.lax.bitcast_convert_type(
        s.astype(jnp.bfloat16), jnp.uint16).astype(jnp.uint32) for s in sig]
    s01_ref[...] = bits[0] | (bits[1] << 16)
    s23_ref[...] = bits[2] | (bits[3] << 16)


def _sigma_stage(coords, code, W1, b1, Wc, W2, b2):
    coords3 = coords.T.reshape(3, _ROWS, 128)
    grid = (_ROWS // _BLK_ROWS,)
    full = lambda a: pl.BlockSpec(a.shape, lambda i: tuple(0 for _ in a.shape))
    b1r = b1.reshape(1, _HIDDEN)
    b2r = b2.reshape(1, 1)
    out_shape = [
        jax.ShapeDtypeStruct((_ROWS, 128), jnp.int32),
        jax.ShapeDtypeStruct((_ROWS, 128), jnp.uint32),
        jax.ShapeDtypeStruct((_ROWS, 128), jnp.uint32),
    ]
    idx, s01, s23 = pl.pallas_call(
        _sigma_body,
        grid=grid,
        in_specs=[
            pl.BlockSpec((3, _BLK_ROWS, 128), lambda i: (0, i, 0)),
            full(code), full(Wc), full(W1), full(b1r), full(W2), full(b2r),
        ],
        out_specs=[
            pl.BlockSpec((_BLK_ROWS, 128), lambda i: (i, 0)),
            pl.BlockSpec((_BLK_ROWS, 128), lambda i: (i, 0)),
            pl.BlockSpec((_BLK_ROWS, 128), lambda i: (i, 0)),
        ],
        out_shape=out_shape,
    )(coords3, code, Wc, W1, b1r, W2, b2r)
    return idx.reshape(-1), s01.reshape(-1), s23.reshape(-1)


def _wid():
    return lax.axis_index("s") * 2 + lax.axis_index("c")


def _iota16():
    return lax.iota(jnp.int32, 16)


def _scan_count_base():
    cnt0, _ = plsc.scan_count(jnp.zeros((16,), jnp.int32))
    return jnp.min(cnt0)


_SC_MESH = functools.partial(
    plsc.VectorSubcoreMesh, core_axis_name="c", subcore_axis_name="s")
_SC_PARAMS = pltpu.CompilerParams(needs_layout_passes=False)


def _route_body(idx_hbm, s01_hbm, s23_hbm,
                rloc_hbm, r01_hbm, r23_hbm, hist_hbm,
                idx_v, s01_v, s23_v, hist_v, lbase_v, rank_v,
                loc_v, o01_v, o23_v, sem):
    w = _wid()
    off = _scan_count_base()
    cp1 = pltpu.async_copy(idx_hbm.at[pl.ds(w * _SPW, _SPW)], idx_v, sem)
    cp2 = pltpu.async_copy(s01_hbm.at[pl.ds(w * _SPW, _SPW)], s01_v, sem)
    cp3 = pltpu.async_copy(s23_hbm.at[pl.ds(w * _SPW, _SPW)], s23_v, sem)
    cp1.wait()
    cp2.wait()
    cp3.wait()
    for c in range(4):
        hist_v[pl.ds(c * 16, 16)] = jnp.zeros((16,), jnp.int32)

    def hbody(i, carry):
        v = idx_v[pl.ds(i * 16, 16)]
        digit = jax.lax.shift_right_logical(v, _BSHIFT)
        cnt, last = plsc.scan_count(digit)
        g = plsc.load_gather(hist_v, [digit])
        crank = g + cnt - off
        rank_v[pl.ds(i * 16, 16)] = crank
        plsc.store_scatter(hist_v, [digit], crank + 1, mask=last)
        return carry

    lax.fori_loop(0, _SPW // 16, hbody, 0)
    cph = pltpu.async_copy(hist_v, hist_hbm.at[w], sem)

    carry = jnp.zeros((), jnp.int32)
    for c in range(4):
        t = hist_v[pl.ds(c * 16, 16)]
        lbase_v[pl.ds(c * 16, 16)] = plsc.cumsum(t) - t + carry
        carry = carry + jnp.sum(t)

    def pbody(i):
        v = idx_v[pl.ds(i * 16, 16)]
        digit = jax.lax.shift_right_logical(v, _BSHIFT)
        g = plsc.load_gather(lbase_v, [digit])
        dest = g + rank_v[pl.ds(i * 16, 16)]
        plsc.store_scatter(loc_v, [dest], v & jnp.int32(_BCELLS - 1))
        plsc.store_scatter(o01_v, [dest],
                           plsc.bitcast(s01_v[pl.ds(i * 16, 16)], jnp.int32))
        plsc.store_scatter(o23_v, [dest],
                           plsc.bitcast(s23_v[pl.ds(i * 16, 16)], jnp.int32))

    plsc.parallel_loop(0, _SPW // 16, 1, unroll=2)(pbody)

    co1 = pltpu.async_copy(loc_v, rloc_hbm.at[pl.ds(w * _SPW, _SPW)], sem)
    co2 = pltpu.async_copy(o01_v, r01_hbm.at[pl.ds(w * _SPW, _SPW)], sem)
    co3 = pltpu.async_copy(o23_v, r23_hbm.at[pl.ds(w * _SPW, _SPW)], sem)
    cph.wait()
    co1.wait()
    co2.wait()
    co3.wait()


def _route_stage(idx, s01, s23):
    k = pl.kernel(
        _route_body,
        out_type=[
            jax.ShapeDtypeStruct((_N + _PAD,), jnp.int32),
            jax.ShapeDtypeStruct((_N + _PAD,), jnp.int32),
            jax.ShapeDtypeStruct((_N + _PAD,), jnp.int32),
            jax.ShapeDtypeStruct((_NW, _NB), jnp.int32),
        ],
        mesh=_SC_MESH(),
        compiler_params=_SC_PARAMS,
        scratch_types=[
            pltpu.VMEM((_SPW,), jnp.int32),
            pltpu.VMEM((_SPW,), jnp.uint32),
            pltpu.VMEM((_SPW,), jnp.uint32),
            pltpu.VMEM((_NB,), jnp.int32),
            pltpu.VMEM((_NB,), jnp.int32),
            pltpu.VMEM((_SPW,), jnp.int32),
            pltpu.VMEM((_SPW,), jnp.int32),
            pltpu.VMEM((_SPW,), jnp.int32),
            pltpu.VMEM((_SPW,), jnp.int32),
            pltpu.SemaphoreType.DMA,
        ],
    )
    return k(idx, s01, s23)


def _extract(row_ref, wp, col):
    iot = _iota16()
    acc = jnp.zeros((), jnp.int32)
    for c in range(4):
        t = row_ref[wp, pl.ds(c * 16, 16)]
        acc = acc + jnp.sum(jnp.where(iot + c * 16 == col, t, 0))
    return acc


def _apply_body(dg_hbm, rloc_hbm, r01_hbm, r23_hbm, hist_hbm,
                out_hbm, part_hbm,
                d0_v, d1_v, tmp_v, sl_v, s01_v, s23_v, hist_v,
                meta_s, acc_v, sem, sem_d, sem_o0, sem_o1):
    w = _wid()
    iot = _iota16()
    pltpu.async_copy(hist_hbm, hist_v, sem_d).wait()

    def packrow(wp, carry):
        c0 = jnp.zeros((), jnp.int32)
        for c in range(4):
            t = hist_v[wp, pl.ds(c * 16, 16)]
            excl = plsc.cumsum(t) - t + c0
            c0 = c0 + jnp.sum(t)
            hist_v[wp, pl.ds(c * 16, 16)] = (excl << 13) | t
        return carry

    lax.fori_loop(0, _NW, packrow, 0)

    def fill0(i, carry):
        for c in range(8):
            tmp_v[pl.ds((i * 8 + c) * 16, 16)] = jnp.zeros((16,), jnp.int32)
        return carry

    lax.fori_loop(0, _BCELLS // 128, fill0, 0)

    acc = jnp.zeros((16,), jnp.float32)
    pending_out = [None, None]
    for h in range(2):
        b = w * 2 + h

        def stage(wp, carry):
            packed = _extract(hist_v, wp, b)
            glen = packed & jnp.int32(8191)
            gstart = wp * _SPW + (packed >> 13)
            astart = pl.multiple_of(gstart & jnp.int32(~7), 8)
            meta_s[2 * wp] = glen
            meta_s[2 * wp + 1] = gstart - astart
            for k in range(_SLOT // 128):
                src = pl.ds(astart + k * 128, 128)
                dst = pl.ds(wp * _SLOT + k * 128, 128)
                pltpu.async_copy(rloc_hbm.at[src], sl_v.at[dst], sem)
                pltpu.async_copy(r01_hbm.at[src], s01_v.at[dst], sem)
                pltpu.async_copy(r23_hbm.at[src], s23_v.at[dst], sem)
            return carry

        lax.fori_loop(0, _NW, stage, 0)

        def drain(i, carry):
            pltpu.make_async_copy(
                rloc_hbm.at[pl.ds(0, 128)], sl_v.at[pl.ds(0, 128)],
                sem).wait()
            return carry

        lax.fori_loop(0, _NW * (_SLOT // 128) * 3, drain, 0)

        def scatter_pass(mode):
            def segs(wp, carry):
                shift = meta_s[2 * wp + 1]
                endp = shift + meta_s[2 * wp]

                def seg(vi, c2):
                    pos = vi * 16 + iot
                    m = (pos >= shift) & (pos < endp)
                    sl = pl.ds(wp * _SLOT + vi * 16, 16)
                    loc = sl_v[sl] & jnp.int32(_BCELLS - 1)
                    if mode == 0:
                        word = s01_v[sl]
                    elif mode == 1:
                        word = s23_v[sl]
                    else:
                        word = jnp.zeros((16,), jnp.int32)
                    plsc.store_scatter(tmp_v, [loc], word, mask=m)
                    return c2

                lax.fori_loop(0, (endp + 15) // 16, seg, 0)
                return carry

            lax.fori_loop(0, _NW, segs, 0)

        for pair in range(2):
            s0, s1 = 2 * pair, 2 * pair + 1
            if pending_out[0] is not None:
                pending_out[0].wait()
                pending_out[1].wait()
                pending_out = [None, None]
            cpd0 = pltpu.async_copy(
                dg_hbm.at[s0, pl.ds(b * _BCELLS, _BCELLS)], d0_v, sem_d)
            cpd1 = pltpu.async_copy(
                dg_hbm.at[s1, pl.ds(b * _BCELLS, _BCELLS)], d1_v, sem_d)
            scatter_pass(pair)
            cpd0.wait()
            cpd1.wait()

            def merge(i, a):
                for c in range(8):
                    slc = pl.ds((i * 8 + c) * 16, 16)
                    word = tmp_v[slc]
                    d0 = d0_v[slc]
                    d1 = d1_v[slc]
                    bits0 = jax.lax.shift_left(word, 16)
                    bits1 = word & jnp.int32(-65536)
                    t0 = plsc.bitcast(bits0, jnp.float32)
                    t1 = plsc.bitcast(bits1, jnp.float32)
                    o0 = jnp.where(bits0 != 0,
                                   jnp.maximum(d0 * jnp.float32(_DECAY), t0),
                                   d0)
                    o1 = jnp.where(bits1 != 0,
                                   jnp.maximum(d1 * jnp.float32(_DECAY), t1),
                                   d1)
                    d0_v[slc] = o0
                    d1_v[slc] = o1
                    a = a + o0 + o1
                return a

            acc = lax.fori_loop(0, _BCELLS // 128, merge, acc)
            pending_out = [
                pltpu.async_copy(
                    d0_v, out_hbm.at[s0, pl.ds(b * _BCELLS, _BCELLS)],
                    sem_o0),
                pltpu.async_copy(
                    d1_v, out_hbm.at[s1, pl.ds(b * _BCELLS, _BCELLS)],
                    sem_o1),
            ]
        if h == 0:
            scatter_pass(2)
    for cp in pending_out:
        if cp is not None:
            cp.wait()
    acc_v[...] = acc
    pltpu.async_copy(acc_v, part_hbm.at[w], sem_d).wait()


def _apply_stage(density_grid, rloc, r01, r23, hist):
    k = pl.kernel(
        _apply_body,
        out_type=[
            jax.ShapeDtypeStruct((_SCENES, _CELLS), jnp.float32),
            jax.ShapeDtypeStruct((_NW, 16), jnp.float32),
        ],
        mesh=_SC_MESH(),
        compiler_params=_SC_PARAMS,
        scratch_types=[
            pltpu.VMEM((_BCELLS,), jnp.float32),
            pltpu.VMEM((_BCELLS,), jnp.float32),
            pltpu.VMEM((_BCELLS,), jnp.int32),
            pltpu.VMEM((_NW * _SLOT,), jnp.int32),
            pltpu.VMEM((_NW * _SLOT,), jnp.int32),
            pltpu.VMEM((_NW * _SLOT,), jnp.int32),
            pltpu.VMEM((_NW, _NB), jnp.int32),
            pltpu.SMEM((2 * _NW,), jnp.int32),
            pltpu.VMEM((16,), jnp.float32),
            pltpu.SemaphoreType.DMA,
            pltpu.SemaphoreType.DMA,
            pltpu.SemaphoreType.DMA,
            pltpu.SemaphoreType.DMA,
        ],
    )
    return k(density_grid, rloc, r01, r23, hist)


def kernel(density_grid, code, W1, b1, Wc, W2, b2, coords):
    idx, s01, s23 = _sigma_stage(coords, code, W1, b1, Wc, W2, b2)
    rloc, r01, r23, hist = _route_stage(idx, s01, s23)
    new_grid, partials = _apply_stage(density_grid, rloc, r01, r23, hist)
    mean_density = jnp.sum(partials) / jnp.float32(_SCENES * _CELLS)
    return new_grid, mean_density
